# baseline (speedup 1.0000x reference)
import functools

import jax
import jax.numpy as jnp
from jax import lax
from jax.experimental import pallas as pl
from jax.experimental.pallas import tpu as pltpu
from jax.experimental.pallas import tpu_sc as plsc

NUM_FEATURES = 100000
EMBED_DIM = 128
BATCH = 4096
N_FIELDS = 26

_INFO = plsc.get_sparse_core_info()
_NC = _INFO.num_cores
_NS = _INFO.num_subcores
_NW = _NC * _NS

_BATCH_PER_W = BATCH // _NW
_NB = 6
_LA = 3


def _sc_gather(idx, table):
    mesh = plsc.VectorSubcoreMesh(core_axis_name="c", subcore_axis_name="s")

    @functools.partial(
        pl.kernel,
        out_type=jax.ShapeDtypeStruct((N_FIELDS * BATCH, EMBED_DIM), jnp.float32),
        mesh=mesh,
        scratch_types=(
            [pltpu.VMEM((N_FIELDS, _BATCH_PER_W), jnp.int32)]
            + [pltpu.VMEM((_BATCH_PER_W, EMBED_DIM), jnp.float32)] * _NB
            + [pltpu.SemaphoreType.DMA] * (2 * _NB)
        ),
    )
    def k(idx_hbm, table_hbm, out_hbm, idx_v, *rest):
        bufs = rest[:_NB]
        gsems = rest[_NB:2 * _NB]
        osems = rest[2 * _NB:]

        wid = lax.axis_index("s") * _NC + lax.axis_index("c")
        base = wid * _BATCH_PER_W
        pltpu.sync_copy(idx_hbm.at[wid], idx_v)

        def gather_start(f, b):
            pltpu.async_copy(table_hbm.at[idx_v.at[f]], bufs[b], gsems[b])

        def gather_wait(b):
            pltpu.make_async_copy(
                table_hbm.at[idx_v.at[0]], bufs[b], gsems[b]
            ).wait()

        def out_start(f, b):
            pltpu.async_copy(
                bufs[b],
                out_hbm.at[pl.ds(f * BATCH + base, _BATCH_PER_W)],
                osems[b],
            )

        def out_wait(b):
            pltpu.make_async_copy(
                bufs[b], out_hbm.at[pl.ds(base, _BATCH_PER_W)], osems[b]
            ).wait()

        out_pending = [False] * _NB

        def drain_out(b):
            if out_pending[b]:
                out_wait(b)
                out_pending[b] = False

        for f in range(min(_LA, N_FIELDS)):
            gather_start(f, f % _NB)
        for f in range(N_FIELDS):
            b = f % _NB
            gather_wait(b)
            out_start(f, b)
            out_pending[b] = True
            nf = f + _LA
            if nf < N_FIELDS:
                bn = nf % _NB
                drain_out(bn)
                gather_start(nf, bn)
        for b in range(_NB):
            drain_out(b)

    return k(idx, table)


def kernel(X, table):
    idx = X.astype(jnp.int32).reshape(_NW, _BATCH_PER_W, N_FIELDS)
    idx = idx.transpose(0, 2, 1)
    out = _sc_gather(idx, table)
    return out.reshape(N_FIELDS, BATCH, EMBED_DIM).transpose(1, 0, 2)


# speedup vs baseline: 1.0008x; 1.0008x over previous
import functools

import jax
import jax.numpy as jnp
from jax import lax
from jax.experimental import pallas as pl
from jax.experimental.pallas import tpu as pltpu
from jax.experimental.pallas import tpu_sc as plsc

NUM_FEATURES = 100000
EMBED_DIM = 128
BATCH = 4096
N_FIELDS = 26

_INFO = plsc.get_sparse_core_info()
_NC = _INFO.num_cores
_NS = _INFO.num_subcores
_NW = _NC * _NS

_BATCH_PER_W = BATCH // _NW
_FPC = 2
_N_CHUNKS = N_FIELDS // _FPC
_ROWS_PER_C = _FPC * _BATCH_PER_W
_NB = 3
_LA = 2


def _sc_gather(idx, table):
    mesh = plsc.VectorSubcoreMesh(core_axis_name="c", subcore_axis_name="s")

    @functools.partial(
        pl.kernel,
        out_type=jax.ShapeDtypeStruct((N_FIELDS * BATCH, EMBED_DIM), jnp.float32),
        mesh=mesh,
        scratch_types=(
            [pltpu.VMEM((N_FIELDS * _BATCH_PER_W,), jnp.int32)]
            + [pltpu.VMEM((_ROWS_PER_C, EMBED_DIM), jnp.float32)] * _NB
            + [pltpu.SemaphoreType.DMA] * (2 * _NB)
        ),
    )
    def k(idx_hbm, table_hbm, out_hbm, idx_v, *rest):
        bufs = rest[:_NB]
        gsems = rest[_NB:2 * _NB]
        osems = rest[2 * _NB:]

        wid = lax.axis_index("s") * _NC + lax.axis_index("c")
        base = wid * _BATCH_PER_W
        pltpu.sync_copy(idx_hbm.at[wid], idx_v)

        def gather_start(c, b):
            pltpu.async_copy(
                table_hbm.at[idx_v.at[pl.ds(c * _ROWS_PER_C, _ROWS_PER_C)]],
                bufs[b], gsems[b],
            )

        def gather_wait(b):
            pltpu.make_async_copy(
                table_hbm.at[idx_v.at[pl.ds(0, _ROWS_PER_C)]], bufs[b], gsems[b]
            ).wait()

        def out_start(c, b):
            for j in range(_FPC):
                pltpu.async_copy(
                    bufs[b].at[pl.ds(j * _BATCH_PER_W, _BATCH_PER_W)],
                    out_hbm.at[pl.ds((c * _FPC + j) * BATCH + base, _BATCH_PER_W)],
                    osems[b],
                )

        def out_wait(b):
            for _ in range(_FPC):
                pltpu.make_async_copy(
                    bufs[b].at[pl.ds(0, _BATCH_PER_W)],
                    out_hbm.at[pl.ds(base, _BATCH_PER_W)],
                    osems[b],
                ).wait()

        out_pending = [False] * _NB

        def drain_out(b):
            if out_pending[b]:
                out_wait(b)
                out_pending[b] = False

        for c in range(min(_LA, _N_CHUNKS)):
            gather_start(c, c % _NB)
        for c in range(_N_CHUNKS):
            b = c % _NB
            gather_wait(b)
            out_start(c, b)
            out_pending[b] = True
            nc = c + _LA
            if nc < _N_CHUNKS:
                bn = nc % _NB
                drain_out(bn)
                gather_start(nc, bn)
        for b in range(_NB):
            drain_out(b)

    return k(idx, table)


def kernel(X, table):
    idx = X.astype(jnp.int32).reshape(_NW, _BATCH_PER_W, N_FIELDS)
    idx = idx.transpose(0, 2, 1).reshape(_NW, N_FIELDS * _BATCH_PER_W)
    out = _sc_gather(idx, table)
    return out.reshape(N_FIELDS, BATCH, EMBED_DIM).transpose(1, 0, 2)


# speedup vs baseline: 1.0024x; 1.0016x over previous
import functools

import jax
import jax.numpy as jnp
from jax import lax
from jax.experimental import pallas as pl
from jax.experimental.pallas import tpu as pltpu
from jax.experimental.pallas import tpu_sc as plsc

NUM_FEATURES = 100000
EMBED_DIM = 128
BATCH = 4096
N_FIELDS = 26

_INFO = plsc.get_sparse_core_info()
_NC = _INFO.num_cores
_NS = _INFO.num_subcores
_NW = _NC * _NS

_BATCH_PER_W = BATCH // _NW
_FPC = 2
_N_CHUNKS = N_FIELDS // _FPC
_ROWS_PER_C = _FPC * _BATCH_PER_W
_NB = 3
_LA = 2


def _sc_gather(idx, table):
    mesh = plsc.VectorSubcoreMesh(core_axis_name="c", subcore_axis_name="s")

    @functools.partial(
        pl.kernel,
        out_type=jax.ShapeDtypeStruct((N_FIELDS * BATCH, EMBED_DIM), jnp.float32),
        mesh=mesh,
        scratch_types=(
            [pltpu.VMEM((N_FIELDS * _BATCH_PER_W,), jnp.int32)]
            + [pltpu.VMEM((_ROWS_PER_C, EMBED_DIM), jnp.float32)] * _NB
            + [pltpu.SemaphoreType.DMA] * (2 * _NB)
        ),
        compiler_params=pltpu.CompilerParams(
            disable_bounds_checks=True,
            disable_semaphore_checks=True,
        ),
    )
    def k(idx_hbm, table_hbm, out_hbm, idx_v, *rest):
        bufs = rest[:_NB]
        gsems = rest[_NB:2 * _NB]
        osems = rest[2 * _NB:]

        wid = lax.axis_index("s") * _NC + lax.axis_index("c")
        base = wid * _BATCH_PER_W
        pltpu.sync_copy(idx_hbm.at[wid], idx_v)

        def gather_start(c, b):
            pltpu.async_copy(
                table_hbm.at[idx_v.at[pl.ds(c * _ROWS_PER_C, _ROWS_PER_C)]],
                bufs[b], gsems[b],
            )

        def gather_wait(b):
            pltpu.make_async_copy(
                table_hbm.at[idx_v.at[pl.ds(0, _ROWS_PER_C)]], bufs[b], gsems[b]
            ).wait()

        def out_start(c, b):
            for j in range(_FPC):
                pltpu.async_copy(
                    bufs[b].at[pl.ds(j * _BATCH_PER_W, _BATCH_PER_W)],
                    out_hbm.at[pl.ds((c * _FPC + j) * BATCH + base, _BATCH_PER_W)],
                    osems[b],
                )

        def out_wait(b):
            for _ in range(_FPC):
                pltpu.make_async_copy(
                    bufs[b].at[pl.ds(0, _BATCH_PER_W)],
                    out_hbm.at[pl.ds(base, _BATCH_PER_W)],
                    osems[b],
                ).wait()

        out_pending = [False] * _NB

        def drain_out(b):
            if out_pending[b]:
                out_wait(b)
                out_pending[b] = False

        for c in range(min(_LA, _N_CHUNKS)):
            gather_start(c, c % _NB)
        for c in range(_N_CHUNKS):
            b = c % _NB
            gather_wait(b)
            out_start(c, b)
            out_pending[b] = True
            nc = c + _LA
            if nc < _N_CHUNKS:
                bn = nc % _NB
                drain_out(bn)
                gather_start(nc, bn)
        for b in range(_NB):
            drain_out(b)

    return k(idx, table)


def kernel(X, table):
    idx = X.astype(jnp.int32).reshape(_NW, _BATCH_PER_W, N_FIELDS)
    idx = idx.transpose(0, 2, 1).reshape(_NW, N_FIELDS * _BATCH_PER_W)
    out = _sc_gather(idx, table)
    return out.reshape(N_FIELDS, BATCH, EMBED_DIM).transpose(1, 0, 2)


# speedup vs baseline: 1.0051x; 1.0027x over previous
import functools

import jax
import jax.numpy as jnp
from jax import lax
from jax.experimental import pallas as pl
from jax.experimental.pallas import tpu as pltpu
from jax.experimental.pallas import tpu_sc as plsc

NUM_FEATURES = 100000
EMBED_DIM = 128
BATCH = 4096
N_FIELDS = 26

_INFO = plsc.get_sparse_core_info()
_NC = _INFO.num_cores
_NS = _INFO.num_subcores
_NW = _NC * _NS

_BATCH_PER_W = BATCH // _NW
_FPC = 2
_N_CHUNKS = N_FIELDS // _FPC
_ROWS_PER_C = _FPC * _BATCH_PER_W
_NB = 3
_LA = 2


def _sc_gather(idx, table):
    mesh = plsc.VectorSubcoreMesh(core_axis_name="c", subcore_axis_name="s")

    @functools.partial(
        pl.kernel,
        out_type=jax.ShapeDtypeStruct((N_FIELDS * BATCH, EMBED_DIM), jnp.float32),
        mesh=mesh,
        scratch_types=(
            [pltpu.VMEM((N_FIELDS * _BATCH_PER_W,), jnp.int32)]
            + [pltpu.VMEM((_ROWS_PER_C, EMBED_DIM), jnp.float32)] * _NB
            + [pltpu.SemaphoreType.DMA] * (2 * _NB)
        ),
        compiler_params=pltpu.CompilerParams(
            disable_bounds_checks=True,
            disable_semaphore_checks=True,
            skip_device_barrier=True,
        ),
    )
    def k(idx_hbm, table_hbm, out_hbm, idx_v, *rest):
        bufs = rest[:_NB]
        gsems = rest[_NB:2 * _NB]
        osems = rest[2 * _NB:]

        wid = lax.axis_index("s") * _NC + lax.axis_index("c")
        base = wid * _BATCH_PER_W
        pltpu.sync_copy(idx_hbm.at[wid], idx_v)

        def gather_start(c, b):
            pltpu.async_copy(
                table_hbm.at[idx_v.at[pl.ds(c * _ROWS_PER_C, _ROWS_PER_C)]],
                bufs[b], gsems[b],
            )

        def gather_wait(b):
            pltpu.make_async_copy(
                table_hbm.at[idx_v.at[pl.ds(0, _ROWS_PER_C)]], bufs[b], gsems[b]
            ).wait()

        def out_start(c, b):
            for j in range(_FPC):
                pltpu.async_copy(
                    bufs[b].at[pl.ds(j * _BATCH_PER_W, _BATCH_PER_W)],
                    out_hbm.at[pl.ds((c * _FPC + j) * BATCH + base, _BATCH_PER_W)],
                    osems[b],
                )

        def out_wait(b):
            for _ in range(_FPC):
                pltpu.make_async_copy(
                    bufs[b].at[pl.ds(0, _BATCH_PER_W)],
                    out_hbm.at[pl.ds(base, _BATCH_PER_W)],
                    osems[b],
                ).wait()

        out_pending = [False] * _NB

        def drain_out(b):
            if out_pending[b]:
                out_wait(b)
                out_pending[b] = False

        for c in range(min(_LA, _N_CHUNKS)):
            gather_start(c, c % _NB)
        for c in range(_N_CHUNKS):
            b = c % _NB
            gather_wait(b)
            out_start(c, b)
            out_pending[b] = True
            nc = c + _LA
            if nc < _N_CHUNKS:
                bn = nc % _NB
                drain_out(bn)
                gather_start(nc, bn)
        for b in range(_NB):
            drain_out(b)

    return k(idx, table)


def kernel(X, table):
    idx = X.astype(jnp.int32).reshape(_NW, _BATCH_PER_W, N_FIELDS)
    idx = idx.transpose(0, 2, 1).reshape(_NW, N_FIELDS * _BATCH_PER_W)
    out = _sc_gather(idx, table)
    return out.reshape(N_FIELDS, BATCH, EMBED_DIM).transpose(1, 0, 2)
